# baseline (speedup 1.0000x reference)
import functools

import jax
import jax.numpy as jnp
from jax import lax
from jax.experimental import pallas as pl
from jax.experimental.pallas import tpu as pltpu
from jax.experimental.pallas import tpu_sc as plsc

NC = 2
NS = 16
NW = NC * NS
SUB = 125
GRP = 16
F = 16

BN = 3128


@functools.lru_cache(maxsize=None)
def _make_hop(N, E):
    EW = E // NW
    NG = EW // (GRP * SUB)
    assert NG * GRP * SUB == EW and EW * NW == E
    NP = ((N + 127) // 128) * 128
    RPT = NP // NS
    NZC = 16
    ZR = RPT // NZC
    assert ZR * NZC == RPT and RPT * NS == NP

    mesh = plsc.VectorSubcoreMesh(core_axis_name="c", subcore_axis_name="s")

    @functools.partial(
        pl.kernel,
        out_type=jax.ShapeDtypeStruct((NC, NP, F), jnp.float32),
        mesh=mesh,
        scratch_types=[
            pltpu.VMEM_SHARED((NP, F), jnp.float32),
            pltpu.VMEM((GRP, SUB), jnp.int32),
            pltpu.VMEM((GRP, SUB), jnp.int32),
            pltpu.VMEM((SUB, F), jnp.float32),
            pltpu.VMEM((SUB, F), jnp.float32),
            pltpu.VMEM((SUB, F), jnp.float32),
            pltpu.VMEM((SUB, F), jnp.float32),
            pltpu.VMEM((ZR, F), jnp.float32),
            pltpu.SemaphoreType.DMA,
            pltpu.SemaphoreType.DMA,
            pltpu.SemaphoreType.DMA,
            pltpu.SemaphoreType.DMA,
            pltpu.SemaphoreType.DMA,
            pltpu.SemaphoreType.DMA,
            pltpu.SemaphoreType.DMA,
            pltpu.SemaphoreType.DMA,
            pltpu.SemaphoreType.DMA,
        ],
        compiler_params=pltpu.CompilerParams(use_tc_tiling_on_sc=False),
    )
    def hop(g_hbm, src_hbm, dst_hbm, ea_hbm, part_hbm,
            acc, src_i, dst_i, ea0, ea1, rw0, rw1, zbuf,
            sg0, sg1, se0, se1, ss0, ss1, si0, si1, sz):
        eab = (ea0, ea1)
        rb = (rw0, rw1)
        sg = (sg0, sg1)
        se = (se0, se1)
        ss = (ss0, ss1)
        c = lax.axis_index("c")
        s = lax.axis_index("s")
        w = s * NC + c
        stripe = s * RPT

        def zb(i, carry):
            zbuf[i] = jnp.zeros((F,), jnp.float32)
            return carry
        lax.fori_loop(0, ZR, zb, None)
        zds = [pltpu.async_copy(zbuf, acc.at[pl.ds(stripe + t * ZR, ZR)], sz)
               for t in range(NZC)]
        for d in zds:
            d.wait()
        plsc.subcore_barrier()

        pltpu.sync_copy(dst_hbm.at[w * NG], dst_i)
        pltpu.async_copy(zbuf.at[pl.ds(0, SUB)], acc.at[dst_i.at[0]], ss[0],
                         add=True)
        pltpu.async_copy(zbuf.at[pl.ds(0, SUB)], acc.at[dst_i.at[1]], ss[1],
                         add=True)

        def grp(gi, carry):
            row = w * NG + gi
            i_d0 = pltpu.async_copy(src_hbm.at[row], src_i, si0)
            i_d1 = pltpu.async_copy(dst_hbm.at[row], dst_i, si1)
            pltpu.make_async_copy(rb[0], acc.at[dst_i.at[0]], ss[0]).wait()
            pltpu.make_async_copy(rb[1], acc.at[dst_i.at[1]], ss[1]).wait()
            i_d0.wait()
            i_d1.wait()
            e_d = [None, None]
            g_d = [None, None]
            s_d = [None, None]
            e_d[0] = pltpu.async_copy(ea_hbm.at[row].at[0], eab[0], se[0])
            g_d[0] = pltpu.async_copy(g_hbm.at[src_i.at[0]], rb[0], sg[0])
            for j in range(GRP):
                b = j % 2
                e_d[b].wait()
                g_d[b].wait()
                if j + 1 < GRP:
                    nb = (j + 1) % 2
                    if s_d[nb] is not None:
                        s_d[nb].wait()
                        s_d[nb] = None
                    e_d[nb] = pltpu.async_copy(ea_hbm.at[row].at[j + 1],
                                               eab[nb], se[nb])
                    g_d[nb] = pltpu.async_copy(g_hbm.at[src_i.at[j + 1]],
                                               rb[nb], sg[nb])

                def scale(i, carry2):
                    rb[b][i] = rb[b][i] * eab[b][i]
                    return carry2
                lax.fori_loop(0, SUB, scale, None, unroll=5)
                s_d[b] = pltpu.async_copy(rb[b], acc.at[dst_i.at[j]], ss[b],
                                          add=True)
            return carry
        lax.fori_loop(0, NG, grp, None)
        pltpu.make_async_copy(rb[0], acc.at[dst_i.at[0]], ss[0]).wait()
        pltpu.make_async_copy(rb[1], acc.at[dst_i.at[1]], ss[1]).wait()
        plsc.subcore_barrier()

        pltpu.sync_copy(acc.at[pl.ds(stripe, RPT)],
                        part_hbm.at[c].at[pl.ds(stripe, RPT)])

    return hop


def _tc_specs(N, n_w):
    grid = (N // BN,)
    part = pl.BlockSpec((NC, BN, F), lambda i: (0, i, 0))
    mat = pl.BlockSpec((BN, F), lambda i: (i, 0))
    wspec = pl.BlockSpec((F, F), lambda i: (0, 0))
    return grid, part, mat, wspec


def _prep(part, x, w0, N):
    def body(p_ref, x_ref, w_ref, dinv_o, dinv2_o, g_o, out_o):
        deg = p_ref[0][:, 0:1] + p_ref[1][:, 0:1]
        dinv = jnp.where(deg > 0, lax.rsqrt(jnp.maximum(deg, 1e-12)), 0.0)
        dinv16 = jnp.broadcast_to(dinv, (BN, F))
        dinv_o[...] = dinv16
        dinv2_o[...] = dinv16 * dinv16
        g_o[...] = x_ref[...] * dinv16
        out_o[...] = jnp.dot(x_ref[...], w_ref[...],
                             preferred_element_type=jnp.float32)
    grid, part_s, mat, wspec = _tc_specs(N, 1)
    return pl.pallas_call(
        body,
        grid=grid,
        in_specs=[part_s, mat, wspec],
        out_specs=[mat, mat, mat, mat],
        out_shape=[jax.ShapeDtypeStruct((N, F), jnp.float32)] * 4,
    )(part, x, w0)


def _gnext(part, dinv2, N):
    def body(p_ref, d2_ref, g_o):
        g_o[...] = (p_ref[0] + p_ref[1]) * d2_ref[...]
    grid, part_s, mat, _ = _tc_specs(N, 1)
    return pl.pallas_call(
        body,
        grid=grid,
        in_specs=[part_s, mat],
        out_specs=mat,
        out_shape=jax.ShapeDtypeStruct((N, F), jnp.float32),
    )(part, dinv2)


def _acc_out(part, dinv, out_in, wk, N):
    def body(p_ref, d_ref, o_ref, w_ref, out_o):
        h = (p_ref[0] + p_ref[1]) * d_ref[...]
        out_o[...] = o_ref[...] + jnp.dot(h, w_ref[...],
                                          preferred_element_type=jnp.float32)
    grid, part_s, mat, wspec = _tc_specs(N, 1)
    return pl.pallas_call(
        body,
        grid=grid,
        in_specs=[part_s, mat, mat, wspec],
        out_specs=mat,
        out_shape=jax.ShapeDtypeStruct((N, F), jnp.float32),
    )(part, dinv, out_in, wk)


def _layer_end(part, dinv, out_in, w4, b, wn0, N):
    def body(p_ref, d_ref, o_ref, w_ref, b_ref, wn_ref, g_o, out_o):
        h = (p_ref[0] + p_ref[1]) * d_ref[...]
        z = o_ref[...] + jnp.dot(h, w_ref[...],
                                 preferred_element_type=jnp.float32) + b_ref[...]
        z = jnp.where(z >= 0, z, 0.01 * z)
        out_o[...] = jnp.dot(z, wn_ref[...],
                             preferred_element_type=jnp.float32)
        g_o[...] = z * d_ref[...]
    grid, part_s, mat, wspec = _tc_specs(N, 2)
    bspec = pl.BlockSpec((1, F), lambda i: (0, 0))
    return pl.pallas_call(
        body,
        grid=grid,
        in_specs=[part_s, mat, mat, wspec, bspec, wspec],
        out_specs=[mat, mat],
        out_shape=[jax.ShapeDtypeStruct((N, F), jnp.float32)] * 2,
    )(part, dinv, out_in, w4, b, wn0)


def _final(part, dinv, out_in, w4, b, wout, bout, N):
    def body(p_ref, d_ref, o_ref, w_ref, b_ref, wo_ref, bo_ref, y_o):
        h = (p_ref[0] + p_ref[1]) * d_ref[...]
        z = o_ref[...] + jnp.dot(h, w_ref[...],
                                 preferred_element_type=jnp.float32) + b_ref[...]
        z = jnp.where(z >= 0, z, 0.01 * z)
        y_o[...] = jnp.dot(z, wo_ref[...],
                           preferred_element_type=jnp.float32) + bo_ref[...]
    grid, part_s, mat, wspec = _tc_specs(N, 1)
    bspec = pl.BlockSpec((1, F), lambda i: (0, 0))
    wospec = pl.BlockSpec((F, 1), lambda i: (0, 0))
    bospec = pl.BlockSpec((1, 1), lambda i: (0, 0))
    yspec = pl.BlockSpec((BN, 1), lambda i: (i, 0))
    return pl.pallas_call(
        body,
        grid=grid,
        in_specs=[part_s, mat, mat, wspec, bspec, wospec, bospec],
        out_specs=yspec,
        out_shape=jax.ShapeDtypeStruct((N, 1), jnp.float32),
    )(part, dinv, out_in, w4, b, wout, bout)


def kernel(x, edge_index, edge_attr, batch, W1, b1, W2, b2, W3, b3, Wout, bout):
    N, _ = x.shape
    E = edge_attr.shape[0]
    ng = E // (GRP * SUB)
    src3 = edge_index[0].reshape(ng, GRP, SUB)
    dst3 = edge_index[1].reshape(ng, GRP, SUB)
    ea = edge_attr.astype(jnp.float32)
    ea3 = jnp.broadcast_to(ea[:, None], (E, F)).reshape(ng, GRP, SUB, F)

    hop = _make_hop(N, E)

    NPAD = ((N + 127) // 128) * 128
    xp = jnp.pad(x, ((0, NPAD - N), (0, 0)))

    def run_hop(feat):
        return hop(feat, src3, dst3, ea3)

    part = run_hop(jnp.ones((NPAD, F), jnp.float32))
    dinv, dinv2, g, out = _prep(part, xp, W1[0], NPAD)

    Ws = (W1, W2, W3)
    bs = (b1, b2, b3)
    y = None
    for li in range(3):
        for k in range(1, 5):
            part = run_hop(g)
            if k < 4:
                g = _gnext(part, dinv2, NPAD)
                out = _acc_out(part, dinv, out, Ws[li][k], NPAD)
            elif li < 2:
                g, out = _layer_end(part, dinv, out, Ws[li][4],
                                    bs[li].reshape(1, F), Ws[li + 1][0], NPAD)
            else:
                y = _final(part, dinv, out, W3[4], b3.reshape(1, F),
                           Wout, bout.reshape(1, 1), NPAD)
    return y[:N]


# speedup vs baseline: 1.0633x; 1.0633x over previous
import functools

import jax
import jax.numpy as jnp
from jax import lax
from jax.experimental import pallas as pl
from jax.experimental.pallas import tpu as pltpu
from jax.experimental.pallas import tpu_sc as plsc

NC = 2
NS = 16
NW = NC * NS
SUB = 125
GRP = 16
F = 16

BN = 3128


@functools.lru_cache(maxsize=None)
def _make_hop(N, E):
    EW = E // NW
    NG = EW // (GRP * SUB)
    assert NG * GRP * SUB == EW and EW * NW == E
    NP = ((N + 127) // 128) * 128
    RPT = NP // NS
    NZC = 16
    ZR = RPT // NZC
    assert ZR * NZC == RPT and RPT * NS == NP

    mesh = plsc.VectorSubcoreMesh(core_axis_name="c", subcore_axis_name="s")

    @functools.partial(
        pl.kernel,
        out_type=jax.ShapeDtypeStruct((NC, NP, F), jnp.float32),
        mesh=mesh,
        scratch_types=[
            pltpu.VMEM_SHARED((NP, F), jnp.float32),
            pltpu.VMEM((GRP, SUB), jnp.int32),
            pltpu.VMEM((GRP, SUB), jnp.int32),
            pltpu.VMEM((SUB, F), jnp.float32),
            pltpu.VMEM((SUB, F), jnp.float32),
            pltpu.VMEM((SUB, F), jnp.float32),
            pltpu.VMEM((SUB, F), jnp.float32),
            pltpu.VMEM((ZR, F), jnp.float32),
            pltpu.SemaphoreType.DMA,
            pltpu.SemaphoreType.DMA,
            pltpu.SemaphoreType.DMA,
            pltpu.SemaphoreType.DMA,
            pltpu.SemaphoreType.DMA,
            pltpu.SemaphoreType.DMA,
            pltpu.SemaphoreType.DMA,
            pltpu.SemaphoreType.DMA,
            pltpu.SemaphoreType.DMA,
        ],
        compiler_params=pltpu.CompilerParams(use_tc_tiling_on_sc=False),
    )
    def hop(g_hbm, src_hbm, dst_hbm, ea_hbm, part_hbm,
            acc, src_i, dst_i, ea0, ea1, rw0, rw1, zbuf,
            sg0, sg1, se0, se1, ss0, ss1, si0, si1, sz):
        eab = (ea0, ea1)
        rb = (rw0, rw1)
        sg = (sg0, sg1)
        se = (se0, se1)
        ss = (ss0, ss1)
        c = lax.axis_index("c")
        s = lax.axis_index("s")
        w = s * NC + c
        stripe = s * RPT

        def zb(i, carry):
            zbuf[i] = jnp.zeros((F,), jnp.float32)
            return carry
        lax.fori_loop(0, ZR, zb, None)
        zds = [pltpu.async_copy(zbuf, acc.at[pl.ds(stripe + t * ZR, ZR)], sz)
               for t in range(NZC)]
        for d in zds:
            d.wait()
        plsc.subcore_barrier()

        pltpu.sync_copy(dst_hbm.at[w * NG], dst_i)
        pltpu.async_copy(zbuf.at[pl.ds(0, SUB)], acc.at[dst_i.at[0]], ss[0],
                         add=True)
        pltpu.async_copy(zbuf.at[pl.ds(0, SUB)], acc.at[dst_i.at[1]], ss[1],
                         add=True)

        def grp(gi, carry):
            row = w * NG + gi
            i_d0 = pltpu.async_copy(src_hbm.at[row], src_i, si0)
            i_d1 = pltpu.async_copy(dst_hbm.at[row], dst_i, si1)
            pltpu.make_async_copy(rb[0], acc.at[dst_i.at[0]], ss[0]).wait()
            pltpu.make_async_copy(rb[1], acc.at[dst_i.at[1]], ss[1]).wait()
            i_d0.wait()
            i_d1.wait()
            e_d = [None, None]
            g_d = [None, None]
            s_d = [None, None]
            e_d[0] = pltpu.async_copy(ea_hbm.at[row].at[0], eab[0], se[0])
            g_d[0] = pltpu.async_copy(g_hbm.at[src_i.at[0]], rb[0], sg[0])
            for j in range(GRP):
                b = j % 2
                e_d[b].wait()
                g_d[b].wait()
                if j + 1 < GRP:
                    nb = (j + 1) % 2
                    if s_d[nb] is not None:
                        s_d[nb].wait()
                        s_d[nb] = None
                    e_d[nb] = pltpu.async_copy(ea_hbm.at[row].at[j + 1],
                                               eab[nb], se[nb])
                    g_d[nb] = pltpu.async_copy(g_hbm.at[src_i.at[j + 1]],
                                               rb[nb], sg[nb])

                def scale(i, carry2):
                    rb[b][i] = rb[b][i] * eab[b][i]
                    return carry2
                lax.fori_loop(0, SUB, scale, None, unroll=5)
                s_d[b] = pltpu.async_copy(rb[b], acc.at[dst_i.at[j]], ss[b],
                                          add=True)
            return carry
        lax.fori_loop(0, NG, grp, None)
        pltpu.make_async_copy(rb[0], acc.at[dst_i.at[0]], ss[0]).wait()
        pltpu.make_async_copy(rb[1], acc.at[dst_i.at[1]], ss[1]).wait()
        plsc.subcore_barrier()

        pltpu.sync_copy(acc.at[pl.ds(stripe, RPT)],
                        part_hbm.at[c].at[pl.ds(stripe, RPT)])

    return hop


@functools.lru_cache(maxsize=None)
def _make_deg(N, E):
    EW = E // NW
    NG = EW // (GRP * SUB)
    NP = ((N + 127) // 128) * 128
    RPT = NP // NS

    mesh = plsc.VectorSubcoreMesh(core_axis_name="c", subcore_axis_name="s")

    @functools.partial(
        pl.kernel,
        out_type=jax.ShapeDtypeStruct((NC, NP), jnp.float32),
        mesh=mesh,
        scratch_types=[
            pltpu.VMEM_SHARED((NP,), jnp.float32),
            pltpu.VMEM((GRP, SUB), jnp.int32),
            pltpu.VMEM((GRP, SUB), jnp.float32),
            pltpu.VMEM((RPT,), jnp.float32),
            pltpu.SemaphoreType.DMA,
        ],
        compiler_params=pltpu.CompilerParams(use_tc_tiling_on_sc=False),
    )
    def deg(dst_hbm, ea_hbm, part_hbm, acc, dst_i, ea_i, zbuf, ss):
        c = lax.axis_index("c")
        s = lax.axis_index("s")
        w = s * NC + c
        stripe = s * RPT

        def zb(i, carry):
            zbuf[pl.ds(i * F, F)] = jnp.zeros((F,), jnp.float32)
            return carry
        lax.fori_loop(0, RPT // F, zb, None)
        pltpu.sync_copy(zbuf, acc.at[pl.ds(stripe, RPT)])
        plsc.subcore_barrier()

        def grp(gi, carry):
            row = w * NG + gi
            pltpu.sync_copy(dst_hbm.at[row], dst_i)
            pltpu.sync_copy(ea_hbm.at[row], ea_i)
            ds_ = [pltpu.async_copy(ea_i.at[j], acc.at[dst_i.at[j]], ss,
                                    add=True)
                   for j in range(GRP)]
            for d in ds_:
                d.wait()
            return carry
        lax.fori_loop(0, NG, grp, None)
        plsc.subcore_barrier()

        pltpu.sync_copy(acc.at[pl.ds(stripe, RPT)],
                        part_hbm.at[c].at[pl.ds(stripe, RPT)])

    return deg


def _tc_specs(N, n_w):
    grid = (N // BN,)
    part = pl.BlockSpec((NC, BN, F), lambda i: (0, i, 0))
    mat = pl.BlockSpec((BN, F), lambda i: (i, 0))
    wspec = pl.BlockSpec((F, F), lambda i: (0, 0))
    return grid, part, mat, wspec


def _prep(degT, x, w0, N):
    def body(p_ref, x_ref, w_ref, dinv_o, g_o, out_o):
        deg = p_ref[:, 0:1] + p_ref[:, 1:2]
        dinv = jnp.where(deg > 0, lax.rsqrt(jnp.maximum(deg, 1e-12)), 0.0)
        dinv16 = jnp.broadcast_to(dinv, (BN, F))
        dinv_o[...] = dinv16
        g_o[...] = x_ref[...] * dinv16
        out_o[...] = jnp.dot(x_ref[...], w_ref[...],
                             preferred_element_type=jnp.float32)
    grid, part_s, mat, wspec = _tc_specs(N, 1)
    degspec = pl.BlockSpec((BN, NC), lambda i: (i, 0))
    return pl.pallas_call(
        body,
        grid=grid,
        in_specs=[degspec, mat, wspec],
        out_specs=[mat, mat, mat],
        out_shape=[jax.ShapeDtypeStruct((N, F), jnp.float32)] * 3,
    )(degT, x, w0)


def _mid(part, dinv, out_in, wk, N):
    def body(p_ref, d_ref, o_ref, w_ref, g_o, out_o):
        h = (p_ref[0] + p_ref[1]) * d_ref[...]
        out_o[...] = o_ref[...] + jnp.dot(h, w_ref[...],
                                          preferred_element_type=jnp.float32)
        g_o[...] = h * d_ref[...]
    grid, part_s, mat, wspec = _tc_specs(N, 1)
    return pl.pallas_call(
        body,
        grid=grid,
        in_specs=[part_s, mat, mat, wspec],
        out_specs=[mat, mat],
        out_shape=[jax.ShapeDtypeStruct((N, F), jnp.float32)] * 2,
    )(part, dinv, out_in, wk)


def _layer_end(part, dinv, out_in, w4, b, wn0, N):
    def body(p_ref, d_ref, o_ref, w_ref, b_ref, wn_ref, g_o, out_o):
        h = (p_ref[0] + p_ref[1]) * d_ref[...]
        z = o_ref[...] + jnp.dot(h, w_ref[...],
                                 preferred_element_type=jnp.float32) + b_ref[...]
        z = jnp.where(z >= 0, z, 0.01 * z)
        out_o[...] = jnp.dot(z, wn_ref[...],
                             preferred_element_type=jnp.float32)
        g_o[...] = z * d_ref[...]
    grid, part_s, mat, wspec = _tc_specs(N, 2)
    bspec = pl.BlockSpec((1, F), lambda i: (0, 0))
    return pl.pallas_call(
        body,
        grid=grid,
        in_specs=[part_s, mat, mat, wspec, bspec, wspec],
        out_specs=[mat, mat],
        out_shape=[jax.ShapeDtypeStruct((N, F), jnp.float32)] * 2,
    )(part, dinv, out_in, w4, b, wn0)


def _final(part, dinv, out_in, w4, b, wout, bout, N):
    def body(p_ref, d_ref, o_ref, w_ref, b_ref, wo_ref, bo_ref, y_o):
        h = (p_ref[0] + p_ref[1]) * d_ref[...]
        z = o_ref[...] + jnp.dot(h, w_ref[...],
                                 preferred_element_type=jnp.float32) + b_ref[...]
        z = jnp.where(z >= 0, z, 0.01 * z)
        y_o[...] = jnp.dot(z, wo_ref[...],
                           preferred_element_type=jnp.float32) + bo_ref[...]
    grid, part_s, mat, wspec = _tc_specs(N, 1)
    bspec = pl.BlockSpec((1, F), lambda i: (0, 0))
    wospec = pl.BlockSpec((F, 1), lambda i: (0, 0))
    bospec = pl.BlockSpec((1, 1), lambda i: (0, 0))
    yspec = pl.BlockSpec((BN, 1), lambda i: (i, 0))
    return pl.pallas_call(
        body,
        grid=grid,
        in_specs=[part_s, mat, mat, wspec, bspec, wospec, bospec],
        out_specs=yspec,
        out_shape=jax.ShapeDtypeStruct((N, 1), jnp.float32),
    )(part, dinv, out_in, w4, b, wout, bout)


def kernel(x, edge_index, edge_attr, batch, W1, b1, W2, b2, W3, b3, Wout, bout):
    N, _ = x.shape
    E = edge_attr.shape[0]
    ng = E // (GRP * SUB)
    src3 = edge_index[0].reshape(ng, GRP, SUB)
    dst3 = edge_index[1].reshape(ng, GRP, SUB)
    ea = edge_attr.astype(jnp.float32)
    ea3 = jnp.broadcast_to(ea[:, None], (E, F)).reshape(ng, GRP, SUB, F)

    hop = _make_hop(N, E)

    NPAD = ((N + 127) // 128) * 128
    xp = jnp.pad(x, ((0, NPAD - N), (0, 0)))

    def run_hop(feat):
        return hop(feat, src3, dst3, ea3)

    ea2 = edge_attr.astype(jnp.float32).reshape(ng, GRP, SUB)
    deg_part = _make_deg(N, E)(dst3, ea2)
    degT = jnp.transpose(deg_part)
    dinv, g, out = _prep(degT, xp, W1[0], NPAD)

    Ws = (W1, W2, W3)
    bs = (b1, b2, b3)
    y = None
    for li in range(3):
        for k in range(1, 5):
            part = run_hop(g)
            if k < 4:
                g, out = _mid(part, dinv, out, Ws[li][k], NPAD)
            elif li < 2:
                g, out = _layer_end(part, dinv, out, Ws[li][4],
                                    bs[li].reshape(1, F), Ws[li + 1][0], NPAD)
            else:
                y = _final(part, dinv, out, W3[4], b3.reshape(1, F),
                           Wout, bout.reshape(1, 1), NPAD)
    return y[:N]


# speedup vs baseline: 1.0908x; 1.0258x over previous
import functools

import jax
import jax.numpy as jnp
from jax import lax
from jax.experimental import pallas as pl
from jax.experimental.pallas import tpu as pltpu
from jax.experimental.pallas import tpu_sc as plsc

NC = 2
NS = 16
NW = NC * NS
SUB = 125
GRP = 32
F = 16

BN = 3128


@functools.lru_cache(maxsize=None)
def _make_hop(N, E):
    EW = E // NW
    NG = EW // (GRP * SUB)
    assert NG * GRP * SUB == EW and EW * NW == E
    NP = ((N + 127) // 128) * 128
    RPT = NP // NS
    NZC = 16
    ZR = RPT // NZC
    assert ZR * NZC == RPT and RPT * NS == NP

    mesh = plsc.VectorSubcoreMesh(core_axis_name="c", subcore_axis_name="s")

    @functools.partial(
        pl.kernel,
        out_type=jax.ShapeDtypeStruct((NC, NP, F), jnp.float32),
        mesh=mesh,
        scratch_types=[
            pltpu.VMEM_SHARED((NP, F), jnp.float32),
            pltpu.VMEM((GRP, SUB), jnp.int32),
            pltpu.VMEM((GRP, SUB), jnp.int32),
            pltpu.VMEM((SUB, F), jnp.float32),
            pltpu.VMEM((SUB, F), jnp.float32),
            pltpu.VMEM((SUB, F), jnp.float32),
            pltpu.VMEM((SUB, F), jnp.float32),
            pltpu.VMEM((ZR, F), jnp.float32),
            pltpu.SemaphoreType.DMA,
            pltpu.SemaphoreType.DMA,
            pltpu.SemaphoreType.DMA,
            pltpu.SemaphoreType.DMA,
            pltpu.SemaphoreType.DMA,
            pltpu.SemaphoreType.DMA,
            pltpu.SemaphoreType.DMA,
            pltpu.SemaphoreType.DMA,
            pltpu.SemaphoreType.DMA,
        ],
        compiler_params=pltpu.CompilerParams(use_tc_tiling_on_sc=False),
    )
    def hop(g_hbm, src_hbm, dst_hbm, ea_hbm, part_hbm,
            acc, src_i, dst_i, ea0, ea1, rw0, rw1, zbuf,
            sg0, sg1, se0, se1, ss0, ss1, si0, si1, sz):
        eab = (ea0, ea1)
        rb = (rw0, rw1)
        sg = (sg0, sg1)
        se = (se0, se1)
        ss = (ss0, ss1)
        c = lax.axis_index("c")
        s = lax.axis_index("s")
        w = s * NC + c
        stripe = s * RPT

        def zb(i, carry):
            zbuf[i] = jnp.zeros((F,), jnp.float32)
            return carry
        lax.fori_loop(0, ZR, zb, None)
        zds = [pltpu.async_copy(zbuf, acc.at[pl.ds(stripe + t * ZR, ZR)], sz)
               for t in range(NZC)]
        for d in zds:
            d.wait()
        plsc.subcore_barrier()

        pltpu.sync_copy(dst_hbm.at[w * NG], dst_i)
        pltpu.async_copy(zbuf.at[pl.ds(0, SUB)], acc.at[dst_i.at[0]], ss[0],
                         add=True)
        pltpu.async_copy(zbuf.at[pl.ds(0, SUB)], acc.at[dst_i.at[1]], ss[1],
                         add=True)

        def grp(gi, carry):
            row = w * NG + gi
            i_d0 = pltpu.async_copy(src_hbm.at[row], src_i, si0)
            i_d1 = pltpu.async_copy(dst_hbm.at[row], dst_i, si1)
            pltpu.make_async_copy(rb[0], acc.at[dst_i.at[0]], ss[0]).wait()
            pltpu.make_async_copy(rb[1], acc.at[dst_i.at[1]], ss[1]).wait()
            i_d0.wait()
            i_d1.wait()
            e_d = [None, None]
            g_d = [None, None]
            s_d = [None, None]
            e_d[0] = pltpu.async_copy(ea_hbm.at[row].at[0], eab[0], se[0])
            g_d[0] = pltpu.async_copy(g_hbm.at[src_i.at[0]], rb[0], sg[0])
            for j in range(GRP):
                b = j % 2
                e_d[b].wait()
                g_d[b].wait()
                if j + 1 < GRP:
                    nb = (j + 1) % 2
                    if s_d[nb] is not None:
                        s_d[nb].wait()
                        s_d[nb] = None
                    e_d[nb] = pltpu.async_copy(ea_hbm.at[row].at[j + 1],
                                               eab[nb], se[nb])
                    g_d[nb] = pltpu.async_copy(g_hbm.at[src_i.at[j + 1]],
                                               rb[nb], sg[nb])

                def scale(i, carry2):
                    rb[b][i] = rb[b][i] * eab[b][i]
                    return carry2
                lax.fori_loop(0, SUB, scale, None, unroll=5)
                s_d[b] = pltpu.async_copy(rb[b], acc.at[dst_i.at[j]], ss[b],
                                          add=True)
            return carry
        lax.fori_loop(0, NG, grp, None)
        pltpu.make_async_copy(rb[0], acc.at[dst_i.at[0]], ss[0]).wait()
        pltpu.make_async_copy(rb[1], acc.at[dst_i.at[1]], ss[1]).wait()
        plsc.subcore_barrier()

        pltpu.sync_copy(acc.at[pl.ds(stripe, RPT)],
                        part_hbm.at[c].at[pl.ds(stripe, RPT)])

    return hop


@functools.lru_cache(maxsize=None)
def _make_deg(N, E):
    EW = E // NW
    NG = EW // (GRP * SUB)
    NP = ((N + 127) // 128) * 128
    RPT = NP // NS

    mesh = plsc.VectorSubcoreMesh(core_axis_name="c", subcore_axis_name="s")

    @functools.partial(
        pl.kernel,
        out_type=jax.ShapeDtypeStruct((NC, NP), jnp.float32),
        mesh=mesh,
        scratch_types=[
            pltpu.VMEM_SHARED((NP,), jnp.float32),
            pltpu.VMEM((GRP, SUB), jnp.int32),
            pltpu.VMEM((GRP, SUB), jnp.float32),
            pltpu.VMEM((RPT,), jnp.float32),
            pltpu.SemaphoreType.DMA,
        ],
        compiler_params=pltpu.CompilerParams(use_tc_tiling_on_sc=False),
    )
    def deg(dst_hbm, ea_hbm, part_hbm, acc, dst_i, ea_i, zbuf, ss):
        c = lax.axis_index("c")
        s = lax.axis_index("s")
        w = s * NC + c
        stripe = s * RPT

        def zb(i, carry):
            zbuf[pl.ds(i * F, F)] = jnp.zeros((F,), jnp.float32)
            return carry
        lax.fori_loop(0, RPT // F, zb, None)
        pltpu.sync_copy(zbuf, acc.at[pl.ds(stripe, RPT)])
        plsc.subcore_barrier()

        def grp(gi, carry):
            row = w * NG + gi
            pltpu.sync_copy(dst_hbm.at[row], dst_i)
            pltpu.sync_copy(ea_hbm.at[row], ea_i)
            ds_ = [pltpu.async_copy(ea_i.at[j], acc.at[dst_i.at[j]], ss,
                                    add=True)
                   for j in range(GRP)]
            for d in ds_:
                d.wait()
            return carry
        lax.fori_loop(0, NG, grp, None)
        plsc.subcore_barrier()

        pltpu.sync_copy(acc.at[pl.ds(stripe, RPT)],
                        part_hbm.at[c].at[pl.ds(stripe, RPT)])

    return deg


def _tc_specs(N, n_w):
    grid = (N // BN,)
    part = pl.BlockSpec((NC, BN, F), lambda i: (0, i, 0))
    mat = pl.BlockSpec((BN, F), lambda i: (i, 0))
    wspec = pl.BlockSpec((F, F), lambda i: (0, 0))
    return grid, part, mat, wspec


def _prep(degT, x, w0, N):
    def body(p_ref, x_ref, w_ref, dinv_o, g_o, out_o):
        deg = p_ref[:, 0:1] + p_ref[:, 1:2]
        dinv = jnp.where(deg > 0, lax.rsqrt(jnp.maximum(deg, 1e-12)), 0.0)
        dinv16 = jnp.broadcast_to(dinv, (BN, F))
        dinv_o[...] = dinv16
        g_o[...] = x_ref[...] * dinv16
        out_o[...] = jnp.dot(x_ref[...], w_ref[...],
                             preferred_element_type=jnp.float32)
    grid, part_s, mat, wspec = _tc_specs(N, 1)
    degspec = pl.BlockSpec((BN, NC), lambda i: (i, 0))
    return pl.pallas_call(
        body,
        grid=grid,
        in_specs=[degspec, mat, wspec],
        out_specs=[mat, mat, mat],
        out_shape=[jax.ShapeDtypeStruct((N, F), jnp.float32)] * 3,
    )(degT, x, w0)


def _mid(part, dinv, out_in, wk, N):
    def body(p_ref, d_ref, o_ref, w_ref, g_o, out_o):
        h = (p_ref[0] + p_ref[1]) * d_ref[...]
        out_o[...] = o_ref[...] + jnp.dot(h, w_ref[...],
                                          preferred_element_type=jnp.float32)
        g_o[...] = h * d_ref[...]
    grid, part_s, mat, wspec = _tc_specs(N, 1)
    return pl.pallas_call(
        body,
        grid=grid,
        in_specs=[part_s, mat, mat, wspec],
        out_specs=[mat, mat],
        out_shape=[jax.ShapeDtypeStruct((N, F), jnp.float32)] * 2,
    )(part, dinv, out_in, wk)


def _layer_end(part, dinv, out_in, w4, b, wn0, N):
    def body(p_ref, d_ref, o_ref, w_ref, b_ref, wn_ref, g_o, out_o):
        h = (p_ref[0] + p_ref[1]) * d_ref[...]
        z = o_ref[...] + jnp.dot(h, w_ref[...],
                                 preferred_element_type=jnp.float32) + b_ref[...]
        z = jnp.where(z >= 0, z, 0.01 * z)
        out_o[...] = jnp.dot(z, wn_ref[...],
                             preferred_element_type=jnp.float32)
        g_o[...] = z * d_ref[...]
    grid, part_s, mat, wspec = _tc_specs(N, 2)
    bspec = pl.BlockSpec((1, F), lambda i: (0, 0))
    return pl.pallas_call(
        body,
        grid=grid,
        in_specs=[part_s, mat, mat, wspec, bspec, wspec],
        out_specs=[mat, mat],
        out_shape=[jax.ShapeDtypeStruct((N, F), jnp.float32)] * 2,
    )(part, dinv, out_in, w4, b, wn0)


def _final(part, dinv, out_in, w4, b, wout, bout, N):
    def body(p_ref, d_ref, o_ref, w_ref, b_ref, wo_ref, bo_ref, y_o):
        h = (p_ref[0] + p_ref[1]) * d_ref[...]
        z = o_ref[...] + jnp.dot(h, w_ref[...],
                                 preferred_element_type=jnp.float32) + b_ref[...]
        z = jnp.where(z >= 0, z, 0.01 * z)
        y_o[...] = jnp.dot(z, wo_ref[...],
                           preferred_element_type=jnp.float32) + bo_ref[...]
    grid, part_s, mat, wspec = _tc_specs(N, 1)
    bspec = pl.BlockSpec((1, F), lambda i: (0, 0))
    wospec = pl.BlockSpec((F, 1), lambda i: (0, 0))
    bospec = pl.BlockSpec((1, 1), lambda i: (0, 0))
    yspec = pl.BlockSpec((BN, 1), lambda i: (i, 0))
    return pl.pallas_call(
        body,
        grid=grid,
        in_specs=[part_s, mat, mat, wspec, bspec, wospec, bospec],
        out_specs=yspec,
        out_shape=jax.ShapeDtypeStruct((N, 1), jnp.float32),
    )(part, dinv, out_in, w4, b, wout, bout)


def kernel(x, edge_index, edge_attr, batch, W1, b1, W2, b2, W3, b3, Wout, bout):
    N, _ = x.shape
    E = edge_attr.shape[0]
    ng = E // (GRP * SUB)
    src3 = edge_index[0].reshape(ng, GRP, SUB)
    dst3 = edge_index[1].reshape(ng, GRP, SUB)
    ea = edge_attr.astype(jnp.float32)
    ea3 = jnp.broadcast_to(ea[:, None], (E, F)).reshape(ng, GRP, SUB, F)

    hop = _make_hop(N, E)

    NPAD = ((N + 127) // 128) * 128
    xp = jnp.pad(x, ((0, NPAD - N), (0, 0)))

    def run_hop(feat):
        return hop(feat, src3, dst3, ea3)

    ea2 = edge_attr.astype(jnp.float32).reshape(ng, GRP, SUB)
    deg_part = _make_deg(N, E)(dst3, ea2)
    degT = jnp.transpose(deg_part)
    dinv, g, out = _prep(degT, xp, W1[0], NPAD)

    Ws = (W1, W2, W3)
    bs = (b1, b2, b3)
    y = None
    for li in range(3):
        for k in range(1, 5):
            part = run_hop(g)
            if k < 4:
                g, out = _mid(part, dinv, out, Ws[li][k], NPAD)
            elif li < 2:
                g, out = _layer_end(part, dinv, out, Ws[li][4],
                                    bs[li].reshape(1, F), Ws[li + 1][0], NPAD)
            else:
                y = _final(part, dinv, out, W3[4], b3.reshape(1, F),
                           Wout, bout.reshape(1, 1), NPAD)
    return y[:N]
